# baseline (speedup 1.0000x reference)
import functools
import jax
import jax.numpy as jnp
from jax import lax
from jax.experimental import pallas as pl
from jax.experimental.pallas import tpu as pltpu
from jax.experimental.pallas import tpu_sc as plsc

CHARGE_DIM = 64
NC, NS = 2, 16
CHUNK = 1024
KSC = 4
NSLOT = 4


def _sc_fill(charge, length, L, D):
    nw = NC * NS
    per_w = (KSC * 8) // nw
    mesh = plsc.VectorSubcoreMesh(core_axis_name="c", subcore_axis_name="s")

    @functools.partial(
        pl.kernel,
        out_type=jax.ShapeDtypeStruct((KSC, D, L), jnp.float32),
        mesh=mesh,
        scratch_types=[
            pltpu.VMEM((16,), jnp.float32),
            pltpu.VMEM((16,), jnp.int32),
            pltpu.VMEM((8, 2 * CHUNK), jnp.float32),
            pltpu.VMEM((8, 128), jnp.float32),
            pltpu.SemaphoreType.DMA,
        ],
        compiler_params=pltpu.CompilerParams(
            use_tc_tiling_on_sc=True, needs_layout_passes=False
        ),
    )
    def k(charge_hbm, length_hbm, out_hbm, chv, lnv, buf, btile, sem):
        wid = lax.axis_index("c") * NS + lax.axis_index("s")
        b = (wid * per_w) // 8

        pltpu.sync_copy(charge_hbm, chv)
        pltpu.sync_copy(length_hbm, lnv)

        lane = lax.iota(jnp.int32, 16)
        sel = lane == b
        my_charge = jnp.sum(jnp.where(sel, chv[...], jnp.float32(0.0)))
        my_len = jnp.sum(jnp.where(sel, lnv[...], jnp.int32(0)))

        full = my_len // 128
        rem = my_len - full * 128

        chvec = lax.broadcast(my_charge, (16,))
        zvec = jnp.zeros((16,), jnp.float32)

        def fill(i, _):
            c = i * 16
            for r in range(8):
                buf[r, pl.ds(c, 16)] = chvec
                buf[r, pl.ds(CHUNK + c, 16)] = zvec
            return 0

        lax.fori_loop(0, CHUNK // 16, fill, 0)

        for g in range(8):
            bval = jnp.where(g * 16 + lane < rem, my_charge, jnp.float32(0.0))
            for r in range(8):
                btile[r, pl.ds(g * 16, 16)] = bval

        copies = []
        for j in range(per_w):
            dt = (wid * per_w + j) % 8
            for c in range(L // CHUNK):
                s = pl.multiple_of(
                    jnp.clip((c + 1) * CHUNK - full * 128, 0, CHUNK), 128
                )
                copies.append(
                    pltpu.async_copy(
                        buf.at[:, pl.ds(s, CHUNK)],
                        out_hbm.at[b, pl.ds(dt * 8, 8), pl.ds(c * CHUNK, CHUNK)],
                        sem,
                    )
                )
        for cp in copies:
            cp.wait()

        @pl.when(full < L // 128)
        def _():
            bcopies = []
            for j in range(per_w):
                dt = (wid * per_w + j) % 8
                bcopies.append(
                    pltpu.async_copy(
                        btile.at[:, :],
                        out_hbm.at[
                            b, pl.ds(dt * 8, 8),
                            pl.ds(pl.multiple_of(full * 128, 128), 128),
                        ],
                        sem,
                    )
                )
            for cp in bcopies:
                cp.wait()

    return k(charge, length)


def _tc_fill(charge, length, n, L, D):
    def body(charge_ref, length_ref, out_ref, buf, sems):
        b = pl.program_id(0)
        slot = lax.rem(b, NSLOT)
        ch = charge_ref[b]
        ln = length_ref[b]

        @pl.when(b >= NSLOT)
        def _():
            pltpu.make_async_copy(
                buf.at[slot], out_ref.at[b - NSLOT], sems.at[slot]
            ).wait()

        pos = lax.broadcasted_iota(jnp.int32, (D, L), 1)
        buf[slot] = jnp.where(pos < ln, ch, jnp.float32(0.0))
        pltpu.make_async_copy(buf.at[slot], out_ref.at[b], sems.at[slot]).start()

        @pl.when(b == n - 1)
        def _():
            for k in range(min(NSLOT, n)):
                r = n - min(NSLOT, n) + k
                pltpu.make_async_copy(
                    buf.at[r % NSLOT], out_ref.at[r], sems.at[r % NSLOT]
                ).wait()

    return pl.pallas_call(
        body,
        grid=(n,),
        in_specs=[
            pl.BlockSpec(memory_space=pltpu.SMEM),
            pl.BlockSpec(memory_space=pltpu.SMEM),
        ],
        out_specs=pl.BlockSpec(memory_space=pl.ANY),
        out_shape=jax.ShapeDtypeStruct((n, D, L), jnp.float32),
        scratch_shapes=[
            pltpu.VMEM((NSLOT, D, L), jnp.float32),
            pltpu.SemaphoreType.DMA((NSLOT,)),
        ],
    )(charge, length)


def kernel(sequence, charge, length):
    B, L = sequence.shape
    D = CHARGE_DIM
    sc_part = _sc_fill(charge, length, L, D)
    tc_part = _tc_fill(charge[KSC:], length[KSC:], B - KSC, L, D)
    out_bdl = jnp.concatenate([sc_part, tc_part], axis=0)
    return out_bdl.transpose(0, 2, 1)


# speedup vs baseline: 6.0187x; 6.0187x over previous
import jax
import jax.numpy as jnp
from jax.experimental import pallas as pl
from jax.experimental.pallas import tpu as pltpu

CHARGE_DIM = 64
NSLOT = 8
HD = 32


def kernel(sequence, charge, length):
    B, L = sequence.shape
    D = CHARGE_DIM
    N = B * (D // HD)

    def body(charge_ref, length_ref, out_ref, buf, sems):
        t = pl.program_id(0)
        b = t // (D // HD)
        half = jax.lax.rem(t, D // HD)
        slot = jax.lax.rem(t, NSLOT)
        ch = charge_ref[b]
        ln = length_ref[b]

        def dst(step):
            return out_ref.at[step // (D // HD),
                              pl.ds(jax.lax.rem(step, D // HD) * HD, HD)]

        @pl.when(t >= NSLOT)
        def _():
            pltpu.make_async_copy(
                buf.at[slot], dst(t - NSLOT), sems.at[slot]
            ).wait()

        pos = jax.lax.broadcasted_iota(jnp.int32, (HD, L), 1)
        buf[slot] = jnp.where(pos < ln, ch, jnp.float32(0.0))
        pltpu.make_async_copy(buf.at[slot], dst(t), sems.at[slot]).start()

        @pl.when(t == N - 1)
        def _():
            for k in range(NSLOT):
                r = N - NSLOT + k
                pltpu.make_async_copy(
                    buf.at[r % NSLOT], dst(r), sems.at[r % NSLOT]
                ).wait()

    out_bdl = pl.pallas_call(
        body,
        grid=(N,),
        in_specs=[
            pl.BlockSpec(memory_space=pltpu.SMEM),
            pl.BlockSpec(memory_space=pltpu.SMEM),
        ],
        out_specs=pl.BlockSpec(memory_space=pl.ANY),
        out_shape=jax.ShapeDtypeStruct((B, D, L), jnp.float32),
        scratch_shapes=[
            pltpu.VMEM((NSLOT, HD, L), jnp.float32),
            pltpu.SemaphoreType.DMA((NSLOT,)),
        ],
    )(charge, length)
    return out_bdl.transpose(0, 2, 1)


# speedup vs baseline: 6.2092x; 1.0317x over previous
import jax
import jax.numpy as jnp
from jax.experimental import pallas as pl
from jax.experimental.pallas import tpu as pltpu

CHARGE_DIM = 64
NSLOT = 4


def kernel(sequence, charge, length):
    B, L = sequence.shape
    D = CHARGE_DIM

    def body(charge_ref, length_ref, out_ref, buf, sems):
        b = pl.program_id(0)
        slot = jax.lax.rem(b, NSLOT)
        ch = charge_ref[b]
        ln = length_ref[b]

        @pl.when(b >= NSLOT)
        def _():
            pltpu.make_async_copy(
                buf.at[slot], out_ref.at[b - NSLOT], sems.at[slot]
            ).wait()

        pos = jax.lax.broadcasted_iota(jnp.int32, (D, L), 1)
        buf[slot] = jnp.where(pos < ln, ch, jnp.float32(0.0))
        pltpu.make_async_copy(buf.at[slot], out_ref.at[b], sems.at[slot]).start()

        @pl.when(b == B - 1)
        def _():
            for k in range(NSLOT):
                r = B - NSLOT + k
                s = r % NSLOT
                pltpu.make_async_copy(
                    buf.at[s], out_ref.at[r], sems.at[s]
                ).wait()

    out_bdl = pl.pallas_call(
        body,
        grid=(B,),
        in_specs=[
            pl.BlockSpec(memory_space=pltpu.SMEM),
            pl.BlockSpec(memory_space=pltpu.SMEM),
        ],
        out_specs=pl.BlockSpec(memory_space=pl.ANY),
        out_shape=jax.ShapeDtypeStruct((B, D, L), jnp.float32),
        scratch_shapes=[
            pltpu.VMEM((NSLOT, D, L), jnp.float32),
            pltpu.SemaphoreType.DMA((NSLOT,)),
        ],
    )(charge, length)
    return out_bdl.transpose(0, 2, 1)


# speedup vs baseline: 6.2164x; 1.0012x over previous
import jax
import jax.numpy as jnp
from jax.experimental import pallas as pl
from jax.experimental.pallas import tpu as pltpu

CHARGE_DIM = 64
NSLOT = 4


def kernel(sequence, charge, length):
    B, L = sequence.shape
    D = CHARGE_DIM

    def body(charge_ref, length_ref, out_ref, buf, sems):
        b = pl.program_id(0)
        slot = jax.lax.rem(b, NSLOT)
        ch = charge_ref[b]
        ln = length_ref[b]

        @pl.when(b >= NSLOT)
        def _():
            pltpu.make_async_copy(
                buf.at[slot], out_ref.at[b - NSLOT], sems.at[slot]
            ).wait()

        pos = jax.lax.broadcasted_iota(jnp.int32, (8, L), 1)
        val8 = jnp.where(pos < ln, ch, jnp.float32(0.0))
        for k in range(D // 8):
            buf[slot, pl.ds(k * 8, 8)] = val8
        pltpu.make_async_copy(buf.at[slot], out_ref.at[b], sems.at[slot]).start()

        @pl.when(b == B - 1)
        def _():
            for k in range(NSLOT):
                r = B - NSLOT + k
                s = r % NSLOT
                pltpu.make_async_copy(
                    buf.at[s], out_ref.at[r], sems.at[s]
                ).wait()

    out_bdl = pl.pallas_call(
        body,
        grid=(B,),
        in_specs=[
            pl.BlockSpec(memory_space=pltpu.SMEM),
            pl.BlockSpec(memory_space=pltpu.SMEM),
        ],
        out_specs=pl.BlockSpec(memory_space=pl.ANY),
        out_shape=jax.ShapeDtypeStruct((B, D, L), jnp.float32),
        scratch_shapes=[
            pltpu.VMEM((NSLOT, D, L), jnp.float32),
            pltpu.SemaphoreType.DMA((NSLOT,)),
        ],
    )(charge, length)
    return out_bdl.transpose(0, 2, 1)
